# baseline (speedup 1.0000x reference)
import functools
import jax
import jax.numpy as jnp
from jax import lax
from jax.experimental import pallas as pl
from jax.experimental.pallas import tpu as pltpu
from jax.experimental.pallas import tpu_sc as plsc

_NC, _NS = 2, 16
_NW = _NC * _NS
_CHUNK = 32
_BLK = 4096
_SC_FRAC_NUM, _SC_FRAC_DEN = 8, 32


def _table_body(emb_ref, comp_ref, w_ref, b_ref, t_ref):
    emb = emb_ref[:]
    w = w_ref[:]
    p = lax.dot_general(emb, w, (((1,), (1,)), ((), ())),
                        preferred_element_type=jnp.float32) + b_ref[:]
    v, h = p.shape
    oh = (comp_ref[:] == lax.broadcasted_iota(jnp.int32, (v, v), 1)
          ).astype(jnp.float32)
    pc = lax.dot_general(oh, p, (((1,), (0,)), ((), ())),
                         preferred_element_type=jnp.float32)
    r = lax.broadcasted_iota(jnp.int32, (h, h), 0)
    c = lax.broadcasted_iota(jnp.int32, (h, h), 1)
    jrev = (r + c == h - 1).astype(jnp.float32)
    pcr = lax.dot_general(pc, jrev, (((1,), (0,)), ((), ())),
                          preferred_element_type=jnp.float32)
    t = jnp.concatenate([p, pcr], axis=1)
    t_ref[:] = jnp.broadcast_to(t[None], t_ref.shape)


def _make_tables(emb_weight, comp2, proj_weight, bias2):
    v, d = emb_weight.shape
    return pl.pallas_call(
        _table_body,
        out_shape=jax.ShapeDtypeStruct((_NW, v, d), jnp.float32),
    )(emb_weight, comp2, proj_weight, bias2).reshape(_NW * v, d)


def _sc_expand(ids_sc, table1, n_sc, n_tok, d, v):
    b_per_w = n_sc // _NW
    n_chunks = b_per_w // _CHUNK
    mesh = plsc.VectorSubcoreMesh(core_axis_name="c", subcore_axis_name="s",
                                  num_cores=_NC, num_subcores=_NS)

    @functools.partial(
        pl.kernel,
        out_type=jax.ShapeDtypeStruct((n_tok * d,), jnp.float32),
        mesh=mesh,
        scratch_types=[
            pltpu.VMEM((b_per_w,), jnp.int32),
            pltpu.VMEM((_CHUNK * d,), jnp.float32),
            pltpu.VMEM((_CHUNK * d,), jnp.float32),
            pltpu.VMEM((v * d,), jnp.float32),
            pltpu.SemaphoreType.DMA,
            pltpu.SemaphoreType.DMA,
        ],
        compiler_params=pltpu.CompilerParams(needs_layout_passes=False),
    )
    def k(ids_hbm, table_hbm, out_hbm, idx_v, rows0, rows1, tab_v, ss0, ss1):
        rows = (rows0, rows1)
        ssems = (ss0, ss1)
        wid = lax.axis_index("s") * _NC + lax.axis_index("c")
        base = wid * b_per_w
        pltpu.sync_copy(table_hbm.at[pl.ds(wid * v * d, v * d)], tab_v)
        pltpu.sync_copy(ids_hbm.at[pl.ds(base, b_per_w)], idx_v)
        lane16 = lax.iota(jnp.int32, 16)

        def scatter_start(cc, b):
            pltpu.async_copy(
                rows[b],
                out_hbm.at[pl.ds((base + cc * _CHUNK) * d, _CHUNK * d)],
                ssems[b])

        def scatter_wait(cc, b):
            pltpu.make_async_copy(
                rows[b],
                out_hbm.at[pl.ds((base + cc * _CHUNK) * d, _CHUNK * d)],
                ssems[b]).wait()

        def expand(cc, b):
            rb = rows[b]

            @pl.loop(0, _CHUNK // 16)
            def _g(g):
                idv = idx_v[pl.ds(cc * _CHUNK + g * 16, 16)] * d

                @pl.loop(0, 16)
                def _t(t):
                    src = jnp.sum(jnp.where(lane16 == t, idv, 0))
                    dst = (g * 16 + t) * d
                    for jj in range(d // 16):
                        rb[pl.ds(dst + 16 * jj, 16)] = (
                            tab_v[pl.ds(src + 16 * jj, 16)])

        for b in range(2):
            expand(b, b)
            scatter_start(b, b)

        @pl.loop(2, n_chunks, step=2)
        def _chunk(c):
            for b in range(2):
                cc = c + b
                scatter_wait(cc - 2, b)
                expand(cc, b)
                scatter_start(cc, b)

        for b in range(2):
            scatter_wait(n_chunks - 2 + b, b)

    return k(ids_sc, table1)


def _expand_body(ids_ref, t_ref, prev_ref, o_ref):
    del prev_ref
    v = t_ref.shape[0]
    oh = (ids_ref[:] == lax.broadcasted_iota(jnp.int32, (_BLK, v), 1)
          ).astype(jnp.float32)
    o_ref[:] = lax.dot_general(oh, t_ref[:], (((1,), (0,)), ((), ())),
                               preferred_element_type=jnp.float32)


def _tc_expand_into(ids2, table, prev, n_sc, n_tok, d, v):
    n_tc = n_tok - n_sc
    blk0 = n_sc // _BLK
    return pl.pallas_call(
        _expand_body,
        grid=(n_tc // _BLK,),
        in_specs=[
            pl.BlockSpec((_BLK, 1), lambda i: (i, 0)),
            pl.BlockSpec((v, d), lambda i: (0, 0)),
            pl.BlockSpec(memory_space=pl.ANY),
        ],
        out_specs=pl.BlockSpec((_BLK, d), lambda i: (i + blk0, 0)),
        out_shape=jax.ShapeDtypeStruct((n_tok, d), jnp.float32),
        input_output_aliases={2: 0},
    )(ids2, table, prev)


def kernel(input_ids, complement_map, emb_weight, proj_weight, proj_bias):
    b, s = input_ids.shape
    v, d = emb_weight.shape
    h = proj_weight.shape[0]
    n_tok = b * s
    n_sc = (n_tok * _SC_FRAC_NUM // _SC_FRAC_DEN) // (_NW * _CHUNK) * (_NW * _CHUNK)
    n_tc = n_tok - n_sc

    comp2 = complement_map.astype(jnp.int32).reshape(v, 1)
    bias2 = proj_bias.astype(jnp.float32).reshape(1, h)
    tables = _make_tables(emb_weight, comp2, proj_weight, bias2)

    ids = input_ids.astype(jnp.int32).reshape(n_tok)
    out_sc = _sc_expand(ids[:n_sc], tables.reshape(_NW * v * d), n_sc,
                        n_tok, d, v)
    out = _tc_expand_into(ids[n_sc:].reshape(n_tc, 1), tables[:v],
                          out_sc.reshape(n_tok, d), n_sc, n_tok, d, v)
    return out.reshape(b, s, d)


# speedup vs baseline: 2.9326x; 2.9326x over previous
import functools
import jax
import jax.numpy as jnp
from jax import lax
from jax.experimental import pallas as pl
from jax.experimental.pallas import tpu as pltpu
from jax.experimental.pallas import tpu_sc as plsc

_NC, _NS = 2, 16
_NW = _NC * _NS
_CHUNK = 32
_BLK = 2048
_SC_FRAC_NUM, _SC_FRAC_DEN = 6, 32


def _table_body(emb_ref, comp_ref, w_ref, b_ref, t_ref):
    emb = emb_ref[:]
    w = w_ref[:]
    p = lax.dot_general(emb, w, (((1,), (1,)), ((), ())),
                        preferred_element_type=jnp.float32) + b_ref[:]
    v, h = p.shape
    oh = (comp_ref[:] == lax.broadcasted_iota(jnp.int32, (v, v), 1)
          ).astype(jnp.float32)
    pc = lax.dot_general(oh, p, (((1,), (0,)), ((), ())),
                         preferred_element_type=jnp.float32)
    r = lax.broadcasted_iota(jnp.int32, (h, h), 0)
    c = lax.broadcasted_iota(jnp.int32, (h, h), 1)
    jrev = (r + c == h - 1).astype(jnp.float32)
    pcr = lax.dot_general(pc, jrev, (((1,), (0,)), ((), ())),
                          preferred_element_type=jnp.float32)
    t = jnp.concatenate([p, pcr], axis=1)
    t_ref[:] = jnp.broadcast_to(t[None], t_ref.shape)


def _make_tables(emb_weight, comp2, proj_weight, bias2):
    v, d = emb_weight.shape
    return pl.pallas_call(
        _table_body,
        out_shape=jax.ShapeDtypeStruct((_NW, v, d), jnp.float32),
    )(emb_weight, comp2, proj_weight, bias2).reshape(_NW * v, d)


def _sc_gather(ids_sc, table, n_sc, n_tok, d, v):
    b_per_w = n_sc // _NW
    n_chunks = b_per_w // _CHUNK
    mesh = plsc.VectorSubcoreMesh(core_axis_name="c", subcore_axis_name="s",
                                  num_cores=_NC, num_subcores=_NS)

    @functools.partial(
        pl.kernel,
        out_type=jax.ShapeDtypeStruct((n_tok, d), jnp.float32),
        mesh=mesh,
        scratch_types=[
            pltpu.VMEM((b_per_w,), jnp.int32),
            pltpu.VMEM((2, _CHUNK, d), jnp.float32),
            pltpu.SemaphoreType.DMA,
            pltpu.SemaphoreType.DMA,
            pltpu.SemaphoreType.DMA,
            pltpu.SemaphoreType.DMA,
        ],
    )
    def k(ids_hbm, table_hbm, out_hbm, idx_v, rows_v, gs0, gs1, ss0, ss1):
        gsems = (gs0, gs1)
        ssems = (ss0, ss1)
        wid = lax.axis_index("s") * _NC + lax.axis_index("c")
        base = wid * b_per_w

        def gather_start(cc, b):
            pltpu.async_copy(table_hbm.at[idx_v.at[pl.ds(cc * _CHUNK, _CHUNK)]],
                             rows_v.at[b], gsems[b])

        def gather_wait(cc, b):
            pltpu.make_async_copy(
                table_hbm.at[idx_v.at[pl.ds(cc * _CHUNK, _CHUNK)]],
                rows_v.at[b], gsems[b]).wait()

        def scatter_start(cc, b):
            pltpu.async_copy(rows_v.at[b],
                             out_hbm.at[pl.ds(base + cc * _CHUNK, _CHUNK)],
                             ssems[b])

        def scatter_wait(cc, b):
            pltpu.make_async_copy(rows_v.at[b],
                                  out_hbm.at[pl.ds(base + cc * _CHUNK,
                                                   _CHUNK)],
                                  ssems[b]).wait()

        pltpu.sync_copy(ids_hbm.at[pl.ds(base, b_per_w)], idx_v)
        off = wid * v

        @pl.loop(0, b_per_w // 16)
        def _off(i):
            sl = pl.ds(i * 16, 16)
            idx_v[sl] = idx_v[sl] + off

        gather_start(0, 0)
        gather_start(1, 1)

        @pl.loop(0, n_chunks - 2, step=2)
        def _chunk(c):
            for b in range(2):
                cc = c + b
                gather_wait(cc, b)
                scatter_start(cc, b)
                scatter_wait(cc, b)
                gather_start(cc + 2, b)

        for b in range(2):
            cc = n_chunks - 2 + b
            gather_wait(cc, b)
            scatter_start(cc, b)
        for b in range(2):
            scatter_wait(n_chunks - 2 + b, b)

    return k(ids_sc, table)


def _expand_body(ids_ref, t_ref, prev_ref, o_ref):
    del prev_ref
    v = t_ref.shape[0]
    oh = (ids_ref[:] == lax.broadcasted_iota(jnp.int32, (_BLK, v), 1)
          ).astype(jnp.float32)
    o_ref[:] = lax.dot_general(oh, t_ref[:], (((1,), (0,)), ((), ())),
                               preferred_element_type=jnp.float32)


def _tc_expand_into(ids2, table, prev, n_sc, n_tok, d, v):
    n_tc = n_tok - n_sc
    blk0 = n_sc // _BLK
    return pl.pallas_call(
        _expand_body,
        grid=(n_tc // _BLK,),
        in_specs=[
            pl.BlockSpec((_BLK, 1), lambda i: (i, 0)),
            pl.BlockSpec((v, d), lambda i: (0, 0)),
            pl.BlockSpec(memory_space=pl.ANY),
        ],
        out_specs=pl.BlockSpec((_BLK, d), lambda i: (i + blk0, 0)),
        out_shape=jax.ShapeDtypeStruct((n_tok, d), jnp.float32),
        input_output_aliases={2: 0},
    )(ids2, table, prev)


def kernel(input_ids, complement_map, emb_weight, proj_weight, proj_bias):
    b, s = input_ids.shape
    v, d = emb_weight.shape
    h = proj_weight.shape[0]
    n_tok = b * s
    n_sc = (n_tok * _SC_FRAC_NUM // _SC_FRAC_DEN) // (_NW * _CHUNK) * (_NW * _CHUNK)
    n_tc = n_tok - n_sc

    comp2 = complement_map.astype(jnp.int32).reshape(v, 1)
    bias2 = proj_bias.astype(jnp.float32).reshape(1, h)
    tables = _make_tables(emb_weight, comp2, proj_weight, bias2)

    ids = input_ids.astype(jnp.int32).reshape(n_tok)
    out_sc = _sc_gather(ids[:n_sc], tables, n_sc, n_tok, d, v)
    out = _tc_expand_into(ids[n_sc:].reshape(n_tc, 1), tables[:v], out_sc,
                          n_sc, n_tok, d, v)
    return out.reshape(b, s, d)
